# baseline (speedup 1.0000x reference)
import functools
import jax
import jax.numpy as jnp
from jax import lax
from jax.experimental import pallas as pl
from jax.experimental.pallas import tpu as pltpu
from jax.experimental.pallas import tpu_sc as plsc

_D = 768
_E = 8
_L = 16
_NKS = _D // _L
_NC = 2
_NS = 16
_NW = _NC * _NS
_CH = 64
_RG = 4


def _sc_gate(rows, x_hbm, w_hbm, o_hbm, xbuf, wbuf, obuf, in_sems, out_sems):
    rw = rows // _NW
    nchunk = rw // _CH
    wid = lax.axis_index("s") * _NC + lax.axis_index("c")
    base = wid * rw

    pltpu.sync_copy(w_hbm, wbuf)

    def in_copy(c, slot):
        return pltpu.make_async_copy(
            x_hbm.at[pl.ds(base + c * _CH, _CH), :], xbuf.at[slot],
            in_sems.at[slot])

    def out_copy(c, slot):
        return pltpu.make_async_copy(
            obuf.at[slot],
            o_hbm.at[pl.ds((base + c * _CH) * _E, _CH * _E)],
            out_sems.at[slot])

    in_copy(0, 0).start()

    def chunk_body(c, carry):
        slot = lax.rem(c, 2)
        nslot = lax.rem(c + 1, 2)

        @pl.when(c + 1 < nchunk)
        def _():
            in_copy(c + 1, nslot).start()

        in_copy(c, slot).wait()

        @pl.when(c >= 2)
        def _():
            out_copy(c - 2, slot).wait()

        lane = lax.iota(jnp.int32, _L)

        def group_body(gi, carry2):
            r0 = gi * _RG
            accs = [[jnp.zeros((_L,), jnp.float32) for _ in range(_E)]
                    for _ in range(_RG)]
            for ks in range(_NKS):
                col = pl.ds(ks * _L, _L)
                for e in range(_E):
                    wv = wbuf[e, col]
                    for j in range(_RG):
                        xv = xbuf[slot, r0 + j, col]
                        accs[j][e] = accs[j][e] + xv * wv
            for p in range(_RG // 2):
                ovec = jnp.zeros((_L,), jnp.float32)
                for h in range(2):
                    for e in range(_E):
                        s = lax.reduce_sum(accs[2 * p + h][e], axes=(0,))
                        ovec = jnp.where(lane == h * _E + e,
                                         jnp.full((_L,), s), ovec)
                obuf[slot, pl.ds((r0 + 2 * p) * _E, _L)] = ovec
            return carry2

        lax.fori_loop(0, _CH // _RG, group_body, 0, unroll=False)
        out_copy(c, slot).start()
        return carry

    lax.fori_loop(0, nchunk, chunk_body, 0, unroll=False)
    out_copy(nchunk - 2, lax.rem(nchunk - 2, 2)).wait()
    out_copy(nchunk - 1, lax.rem(nchunk - 1, 2)).wait()


def sc_gate_call(x, W, rows):
    mesh = plsc.VectorSubcoreMesh(core_axis_name="c", subcore_axis_name="s")
    f = pl.kernel(
        functools.partial(_sc_gate, rows),
        mesh=mesh,
        out_type=jax.ShapeDtypeStruct((rows * _E,), jnp.float32),
        scratch_types=[
            pltpu.VMEM((2, _CH, _D), jnp.float32),
            pltpu.VMEM((_E, _D), jnp.float32),
            pltpu.VMEM((2, _CH * _E), jnp.float32),
            pltpu.SemaphoreType.DMA((2,)),
            pltpu.SemaphoreType.DMA((2,)),
        ],
        compiler_params=pltpu.CompilerParams(needs_layout_passes=False),
    )
    return f(x, W).reshape(rows, _E)


def kernel(x, W):
    return sc_gate_call(x, W, x.shape[0])


# speedup vs baseline: 6.2060x; 6.2060x over previous
import jax
import jax.numpy as jnp
from jax import lax
from jax.experimental import pallas as pl
from jax.experimental.pallas import tpu as pltpu

_ROWS = 32768
_D = 768
_E = 8
_BLOCK_ROWS = 4096


def _gate_body(x_ref, w_ref, o_ref):
    o_ref[...] = lax.dot_general(
        w_ref[...], x_ref[...],
        dimension_numbers=(((1,), (1,)), ((), ())),
        preferred_element_type=jnp.float32)


def kernel(x, W):
    grid = (_ROWS // _BLOCK_ROWS,)
    out_t = pl.pallas_call(
        _gate_body,
        grid=grid,
        in_specs=[
            pl.BlockSpec((_BLOCK_ROWS, _D), lambda i: (i, 0)),
            pl.BlockSpec((_E, _D), lambda i: (0, 0)),
        ],
        out_specs=pl.BlockSpec((_E, _BLOCK_ROWS), lambda i: (0, i)),
        out_shape=jax.ShapeDtypeStruct((_E, _ROWS), jnp.float32),
        compiler_params=pltpu.CompilerParams(
            dimension_semantics=("arbitrary",),
            fuse_transposed_lhs_in_matmul=True,
        ),
    )(x, W)
    return out_t.T
